# baseline (speedup 1.0000x reference)
import functools

import jax
import jax.numpy as jnp
from jax import lax
from jax.experimental import pallas as pl
from jax.experimental.pallas import tpu as pltpu
from jax.experimental.pallas import tpu_sc as plsc

NUM_FIELDS = 26
VOCAB = 100000
EMB_DIM = 32
BATCH = 16384

_INFO = plsc.get_sparse_core_info()
_NC, _NS, _L = _INFO.num_cores, _INFO.num_subcores, _INFO.num_lanes
_NW = _NC * _NS
_HALF = BATCH // 2
_VW = VOCAB // 2


def _sc_embedding_t(xt, tabi):
    mesh = plsc.VectorSubcoreMesh(core_axis_name="c", subcore_axis_name="s")

    @functools.partial(
        pl.kernel,
        mesh=mesh,
        out_type=jax.ShapeDtypeStruct((NUM_FIELDS * EMB_DIM, BATCH),
                                      jnp.float32),
        scratch_types=[
            pltpu.VMEM((2, _VW), jnp.int32),
            pltpu.VMEM((_HALF,), jnp.int32),
            pltpu.VMEM((2, _HALF), jnp.float32),
            pltpu.SemaphoreType.DMA,
            pltpu.SemaphoreType.DMA,
            pltpu.SemaphoreType.DMA,
        ],
        compiler_params=pltpu.CompilerParams(use_tc_tiling_on_sc=False,
                                             needs_layout_passes=False),
    )
    def k(xt_hbm, tab_hbm, out_hbm, vocab_v, idx_v, outr_v,
          vsem0, vsem1, wsem):
        d = lax.axis_index("s") * _NC + lax.axis_index("c")

        def drain_vocab(slot, sem):
            pltpu.make_async_copy(tab_hbm.at[0], vocab_v.at[slot],
                                  sem).wait()

        def drain_write(slot):
            pltpu.make_async_copy(xt_hbm.at[0, pl.ds(0, _HALF)],
                                  outr_v.at[slot], wsem).wait()

        def stage_vocab(f, slot, sem):
            pltpu.async_copy(tab_hbm.at[f * EMB_DIM + d], vocab_v.at[slot],
                             sem)

        def field(g, f, slot, sem):
            row = f * EMB_DIM + d
            drain_vocab(slot, sem)
            for h in range(2):
                pltpu.sync_copy(xt_hbm.at[f, pl.ds(h * _HALF, _HALF)], idx_v)

                if slot > 0:
                    drain_write(h)
                else:
                    @pl.when(g > 0)
                    def _():
                        drain_write(h)

                def gbody(j, c):
                    sl = pl.ds(j * _L, _L)
                    iv = idx_v[sl]
                    hi = iv >= _VW
                    w = plsc.load_gather(
                        vocab_v.at[slot],
                        [iv - jnp.where(hi, _VW, 0)])
                    sh = jnp.where(hi, 16, 0)
                    bits = jnp.bitwise_and(
                        lax.shift_right_logical(w, sh), 0xFFFF)
                    outr_v[h, sl] = plsc.bitcast(
                        lax.shift_left(bits, 16), jnp.float32)
                    return c

                lax.fori_loop(0, _HALF // _L, gbody, 0)
                pltpu.async_copy(outr_v.at[h],
                                 out_hbm.at[row, pl.ds(h * _HALF, _HALF)],
                                 wsem)
            @pl.when(f + 2 < NUM_FIELDS)
            def _():
                stage_vocab(f + 2, slot, sem)

        stage_vocab(0, 0, vsem0)
        stage_vocab(1, 1, vsem1)

        def pbody(g, carry):
            field(g, 2 * g, 0, vsem0)
            field(g, 2 * g + 1, 1, vsem1)
            return carry

        lax.fori_loop(0, NUM_FIELDS // 2, pbody, 0)
        drain_write(0)
        drain_write(1)

    return k(xt, tabi)


def kernel(x_cat, tables):
    xt = x_cat.T.astype(jnp.int32)
    tabt = tables.transpose(0, 2, 1).reshape(
        NUM_FIELDS * EMB_DIM, VOCAB)
    bits = jax.lax.bitcast_convert_type(tabt, jnp.int32)
    tabi = jnp.bitwise_or(
        lax.shift_right_logical(bits[:, :_VW], 16),
        jnp.bitwise_and(bits[:, _VW:], jnp.int32(-65536)))
    out_t = _sc_embedding_t(xt, tabi)
    return out_t.T.reshape(BATCH, NUM_FIELDS * EMB_DIM)


# speedup vs baseline: 1.0194x; 1.0194x over previous
import functools

import jax
import jax.numpy as jnp
from jax import lax
from jax.experimental import pallas as pl
from jax.experimental.pallas import tpu as pltpu
from jax.experimental.pallas import tpu_sc as plsc

NUM_FIELDS = 26
VOCAB = 100000
EMB_DIM = 32
BATCH = 16384

_INFO = plsc.get_sparse_core_info()
_NC, _NS, _L = _INFO.num_cores, _INFO.num_subcores, _INFO.num_lanes
_NW = _NC * _NS
_HALF = BATCH // 2


def _sc_embedding_t(xt, tabt):
    mesh = plsc.VectorSubcoreMesh(core_axis_name="c", subcore_axis_name="s")

    @functools.partial(
        pl.kernel,
        mesh=mesh,
        out_type=jax.ShapeDtypeStruct((NUM_FIELDS * EMB_DIM, BATCH),
                                      jnp.float32),
        scratch_types=[
            pltpu.VMEM((VOCAB,), jnp.float32),
            pltpu.VMEM((_HALF,), jnp.int32),
            pltpu.VMEM((2, _HALF), jnp.float32),
            pltpu.SemaphoreType.DMA,
            pltpu.SemaphoreType.DMA,
        ],
        compiler_params=pltpu.CompilerParams(use_tc_tiling_on_sc=False,
                                             needs_layout_passes=False),
    )
    def k(xt_hbm, tab_hbm, out_hbm, vocab_v, idx_v, outr_v, gsem, wsem):
        d = lax.axis_index("s") * _NC + lax.axis_index("c")

        def drain_write(slot):
            pltpu.make_async_copy(xt_hbm.at[0, pl.ds(0, _HALF)],
                                  outr_v.at[slot], wsem).wait()

        def fbody(f, carry):
            row = f * EMB_DIM + d
            pltpu.sync_copy(tab_hbm.at[row], vocab_v)
            for h in range(2):
                pltpu.sync_copy(xt_hbm.at[f, pl.ds(h * _HALF, _HALF)], idx_v)

                @pl.when(f > 0)
                def _():
                    drain_write(h)

                def gbody(j, c):
                    sl = pl.ds(j * _L, _L)
                    iv = idx_v[sl]
                    outr_v[h, sl] = plsc.load_gather(vocab_v, [iv])
                    return c

                lax.fori_loop(0, _HALF // _L, gbody, 0)
                pltpu.async_copy(outr_v.at[h],
                                 out_hbm.at[row, pl.ds(h * _HALF, _HALF)],
                                 wsem)
            return carry

        lax.fori_loop(0, NUM_FIELDS, fbody, 0)
        drain_write(0)
        drain_write(1)

    return k(xt, tabt)


def kernel(x_cat, tables):
    xt = x_cat.T.astype(jnp.int32)
    tabt = tables.transpose(0, 2, 1).reshape(
        NUM_FIELDS * EMB_DIM, VOCAB)
    out_t = _sc_embedding_t(xt, tabt)
    return out_t.T.reshape(BATCH, NUM_FIELDS * EMB_DIM)
